# baseline (speedup 1.0000x reference)
import functools

import jax
import jax.numpy as jnp
from jax import lax
from jax.experimental import pallas as pl
from jax.experimental.pallas import tpu as pltpu
from jax.experimental.pallas import tpu_sc as plsc

E = 8
K = 2
D = 768
F = 3072
T = 2048
BM = 256
NT = 24
P = NT * BM
NC = 2
NS = 16
NW = NC * NS

_NEG = -1e30


def _router_body(x_ref, wg_ref, pos_ref, w_ref, meta_ref, oh_ref, cum_ref):
    xl = x_ref[...]
    wg = wg_ref[...]
    logits = jnp.dot(xl, wg, preferred_element_type=jnp.float32)
    col = lax.broadcasted_iota(jnp.int32, (T, 128), 1)
    lg = jnp.where(col < E, logits, _NEG)
    m1 = jnp.max(lg, axis=1, keepdims=True)
    a1 = jnp.min(jnp.where(lg == m1, col, 128), axis=1, keepdims=True)
    lg2 = jnp.where(col == a1, _NEG, lg)
    m2 = jnp.max(lg2, axis=1, keepdims=True)
    a2 = jnp.min(jnp.where(lg2 == m2, col, 128), axis=1, keepdims=True)
    w1v = 1.0 / (1.0 + jnp.exp(m2 - m1))
    w2v = 1.0 - w1v

    oh_ref[...] = ((col == a1) | (col == a2)).astype(jnp.float32)
    ri = lax.broadcasted_iota(jnp.int32, (128, 128), 0)
    ci = lax.broadcasted_iota(jnp.int32, (128, 128), 1)
    lower = (ri > ci).astype(jnp.float32)
    carry = jnp.zeros((1, 128), jnp.float32)
    for c in range(T // 128):
        chunk = oh_ref[pl.ds(c * 128, 128), :]
        cum_ref[pl.ds(c * 128, 128), :] = (
            jnp.dot(lower, chunk, preferred_element_type=jnp.float32) + carry)
        carry = carry + jnp.sum(chunk, axis=0, keepdims=True)

    cnt = carry.astype(jnp.int32)
    q = (cnt + (BM - 1)) // BM
    upper = (ri < ci).astype(jnp.float32)
    offq = jnp.dot(q.astype(jnp.float32), upper,
                   preferred_element_type=jnp.float32)
    off = offq * float(BM)
    na = jnp.sum(q)

    cum = cum_ref[...]
    offb = jnp.broadcast_to(off, (T, 128))
    p1 = jnp.sum(jnp.where(col == a1, cum + offb, 0.0), axis=1, keepdims=True)
    p2 = jnp.sum(jnp.where(col == a2, cum + offb, 0.0), axis=1, keepdims=True)

    col8 = lax.broadcasted_iota(jnp.int32, (T, 8), 1)
    pos_ref[...] = jnp.where(col8 == 0, p1.astype(jnp.int32),
                             jnp.where(col8 == 1, p2.astype(jnp.int32), 0))
    w_ref[...] = jnp.where(col8 == 0, w1v, jnp.where(col8 == 1, w2v, 0.0))

    r32 = lax.broadcasted_iota(jnp.int32, (32, 128), 0)
    c32 = lax.broadcasted_iota(jnp.int32, (32, 128), 1)
    starts = (r32 * BM).astype(jnp.float32)
    off32 = jnp.broadcast_to(off, (32, 128))
    ind = ((starts >= off32) & (c32 >= 1) & (c32 < E)).astype(jnp.int32)
    et = jnp.sum(ind, axis=1, keepdims=True)
    meta_ref[...] = jnp.where(r32 == NT, na, jnp.broadcast_to(et, (32, 128)))


def _router(x, wg_pad, interpret=False):
    return pl.pallas_call(
        _router_body,
        out_shape=(
            jax.ShapeDtypeStruct((T, 8), jnp.int32),
            jax.ShapeDtypeStruct((T, 8), jnp.float32),
            jax.ShapeDtypeStruct((32, 128), jnp.int32),
        ),
        scratch_shapes=[
            pltpu.VMEM((T, 128), jnp.float32),
            pltpu.VMEM((T, 128), jnp.float32),
        ],
        interpret=interpret,
    )(x, wg_pad)


_PPW = (K * T) // NW


def _dispatch_body(pos_hbm, w_hbm, x_hbm, xs_hbm, wso_hbm,
                   posb, wvb, rowb, sem, sem2):
    c = lax.axis_index("c")
    s = lax.axis_index("s")
    w = s * NC + c
    pltpu.sync_copy(pos_hbm.at[w], posb)
    pltpu.sync_copy(w_hbm.at[w], wvb)
    xrow = (w & (NS - 1)) * _PPW
    pltpu.sync_copy(x_hbm.at[pl.ds(xrow, _PPW)], rowb)
    cp1 = pltpu.async_copy(rowb, xs_hbm.at[posb], sem)
    cp2 = pltpu.async_copy(wvb, wso_hbm.at[posb], sem2)
    cp1.wait()
    cp2.wait()


@functools.lru_cache(maxsize=None)
def _dispatch_kernel():
    return functools.partial(
        pl.kernel,
        out_type=(
            jax.ShapeDtypeStruct((P, D), jnp.float32),
            jax.ShapeDtypeStruct((P,), jnp.float32),
        ),
        mesh=plsc.VectorSubcoreMesh(core_axis_name="c", subcore_axis_name="s"),
        scratch_types=(
            pltpu.VMEM((_PPW,), jnp.int32),
            pltpu.VMEM((_PPW,), jnp.float32),
            pltpu.VMEM((_PPW, D), jnp.float32),
            pltpu.SemaphoreType.DMA,
            pltpu.SemaphoreType.DMA,
        ),
    )(_dispatch_body)


def _ffn_body(sp_ref, xs_ref, w1_ref, b1_ref, w2_ref, b2_ref, ws_ref, out_ref):
    i = pl.program_id(0)
    na = sp_ref[NT]

    @pl.when(i < na)
    def _():
        xb = xs_ref[...].astype(jnp.bfloat16)
        h = jnp.dot(xb, w1_ref[0], preferred_element_type=jnp.float32)
        h = jnp.maximum(h + b1_ref[0], 0.0).astype(jnp.bfloat16)
        y = jnp.dot(h, w2_ref[0], preferred_element_type=jnp.float32)
        y = y + b2_ref[0]
        out_ref[...] = y * ws_ref[0]


def _ffn(sp, xs, W1, b1, W2, b2, wsr, interpret=False):
    grid_spec = pltpu.PrefetchScalarGridSpec(
        num_scalar_prefetch=1,
        grid=(NT,),
        in_specs=[
            pl.BlockSpec((BM, D), lambda i, sp: (i, 0)),
            pl.BlockSpec((1, D, F), lambda i, sp: (sp[i], 0, 0)),
            pl.BlockSpec((1, 1, F), lambda i, sp: (sp[i], 0, 0)),
            pl.BlockSpec((1, F, D), lambda i, sp: (sp[i], 0, 0)),
            pl.BlockSpec((1, 1, D), lambda i, sp: (sp[i], 0, 0)),
            pl.BlockSpec((1, BM, 1), lambda i, sp: (i, 0, 0)),
        ],
        out_specs=pl.BlockSpec((BM, D), lambda i, sp: (i, 0)),
    )
    return pl.pallas_call(
        _ffn_body,
        grid_spec=grid_spec,
        out_shape=jax.ShapeDtypeStruct((P, D), jnp.float32),
        interpret=interpret,
    )(sp, xs, W1.astype(jnp.bfloat16), b1.reshape(E, 1, F),
      W2.astype(jnp.bfloat16), b2.reshape(E, 1, D), wsr)


_TPW = T // NW


def _combine_body(ys_hbm, p0_hbm, p1_hbm, out_hbm, i0b, i1b, b0, b1, sem):
    c = lax.axis_index("c")
    s = lax.axis_index("s")
    w = s * NC + c
    pltpu.sync_copy(p0_hbm.at[w], i0b)
    pltpu.async_copy(ys_hbm.at[i0b], b0, sem).wait()
    pltpu.sync_copy(p1_hbm.at[w], i1b)
    pltpu.async_copy(ys_hbm.at[i1b], b1, sem).wait()

    def addrow(r, carry):
        for cc in range(D // 16):
            b0[r, pl.ds(cc * 16, 16)] = (
                b0[r, pl.ds(cc * 16, 16)] + b1[r, pl.ds(cc * 16, 16)])
        return carry

    lax.fori_loop(0, _TPW, addrow, 0)
    pltpu.sync_copy(b0, out_hbm.at[pl.ds(w * _TPW, _TPW)])


@functools.lru_cache(maxsize=None)
def _combine_kernel():
    return functools.partial(
        pl.kernel,
        out_type=jax.ShapeDtypeStruct((T, D), jnp.float32),
        mesh=plsc.VectorSubcoreMesh(core_axis_name="c", subcore_axis_name="s"),
        scratch_types=(
            pltpu.VMEM((_TPW,), jnp.int32),
            pltpu.VMEM((_TPW,), jnp.int32),
            pltpu.VMEM((_TPW, D), jnp.float32),
            pltpu.VMEM((_TPW, D), jnp.float32),
            pltpu.SemaphoreType.DMA,
        ),
    )(_combine_body)


def kernel(x, Wg, W1, b1, W2, b2):
    wg_pad = jnp.pad(Wg, ((0, 0), (0, 128 - E)))
    pos, wts, meta = _router(x, wg_pad)
    pos_flat = jnp.concatenate([pos[:, 0], pos[:, 1]]).reshape(K * T // 128, 128)
    w_flat = jnp.concatenate([wts[:, 0], wts[:, 1]]).reshape(K * T // 128, 128)
    sp = meta[:NT + 1, 0]

    xs, wso = _dispatch_kernel()(pos_flat, w_flat, x)
    ys = _ffn(sp, xs, W1, b1, W2, b2, wso.reshape(NT, BM, 1))
    out = _combine_kernel()(
        ys, pos[:, 0].reshape(NW, _TPW), pos[:, 1].reshape(NW, _TPW))
    return out


# speedup vs baseline: 1.1301x; 1.1301x over previous
import functools

import jax
import jax.numpy as jnp
from jax import lax
from jax.experimental import pallas as pl
from jax.experimental.pallas import tpu as pltpu
from jax.experimental.pallas import tpu_sc as plsc

E = 8
K = 2
D = 768
F = 3072
T = 2048
BM = 256
NT = 24
P = NT * BM
NC = 2
NS = 16
NW = NC * NS

_NEG = -1e30


def _router_body(x_ref, wg_ref, pos_ref, w_ref, meta_ref, oh_ref, cum_ref):
    xl = x_ref[...]
    wg = wg_ref[...]
    logits = jnp.dot(xl, wg, preferred_element_type=jnp.float32)
    col = lax.broadcasted_iota(jnp.int32, (T, 128), 1)
    lg = jnp.where(col < E, logits, _NEG)
    m1 = jnp.max(lg, axis=1, keepdims=True)
    a1 = jnp.min(jnp.where(lg == m1, col, 128), axis=1, keepdims=True)
    lg2 = jnp.where(col == a1, _NEG, lg)
    m2 = jnp.max(lg2, axis=1, keepdims=True)
    a2 = jnp.min(jnp.where(lg2 == m2, col, 128), axis=1, keepdims=True)
    w1v = 1.0 / (1.0 + jnp.exp(m2 - m1))
    w2v = 1.0 - w1v

    oh_ref[...] = ((col == a1) | (col == a2)).astype(jnp.float32)
    ri = lax.broadcasted_iota(jnp.int32, (128, 128), 0)
    ci = lax.broadcasted_iota(jnp.int32, (128, 128), 1)
    lower = (ri > ci).astype(jnp.float32)
    carry = jnp.zeros((1, 128), jnp.float32)
    for c in range(T // 128):
        chunk = oh_ref[pl.ds(c * 128, 128), :]
        cum_ref[pl.ds(c * 128, 128), :] = (
            jnp.dot(lower, chunk, preferred_element_type=jnp.float32) + carry)
        carry = carry + jnp.sum(chunk, axis=0, keepdims=True)

    cnt = carry.astype(jnp.int32)
    q = (cnt + (BM - 1)) // BM
    upper = (ri < ci).astype(jnp.float32)
    offq = jnp.dot(q.astype(jnp.float32), upper,
                   preferred_element_type=jnp.float32)
    off = offq * float(BM)
    na = jnp.sum(q)

    cum = cum_ref[...]
    offb = jnp.broadcast_to(off, (T, 128))
    p1 = jnp.sum(jnp.where(col == a1, cum + offb, 0.0), axis=1, keepdims=True)
    p2 = jnp.sum(jnp.where(col == a2, cum + offb, 0.0), axis=1, keepdims=True)

    col8 = lax.broadcasted_iota(jnp.int32, (T, 8), 1)
    pos_ref[...] = jnp.where(col8 == 0, p1.astype(jnp.int32),
                             jnp.where(col8 == 1, p2.astype(jnp.int32), 0))
    w_ref[...] = jnp.where(col8 == 0, w1v, jnp.where(col8 == 1, w2v, 0.0))

    r32 = lax.broadcasted_iota(jnp.int32, (32, 128), 0)
    c32 = lax.broadcasted_iota(jnp.int32, (32, 128), 1)
    starts = (r32 * BM).astype(jnp.float32)
    off32 = jnp.broadcast_to(off, (32, 128))
    ind = ((starts >= off32) & (c32 >= 1) & (c32 < E)).astype(jnp.int32)
    et = jnp.sum(ind, axis=1, keepdims=True)
    meta_ref[...] = jnp.where(r32 == NT, na, jnp.broadcast_to(et, (32, 128)))


def _router(x, wg_pad, interpret=False):
    return pl.pallas_call(
        _router_body,
        out_shape=(
            jax.ShapeDtypeStruct((T, 8), jnp.int32),
            jax.ShapeDtypeStruct((T, 8), jnp.float32),
            jax.ShapeDtypeStruct((32, 128), jnp.int32),
        ),
        scratch_shapes=[
            pltpu.VMEM((T, 128), jnp.float32),
            pltpu.VMEM((T, 128), jnp.float32),
        ],
        interpret=interpret,
    )(x, wg_pad)


_PPW = (K * T) // NW


def _dispatch_body(pos_hbm, w_hbm, x_hbm, xs_hbm, wso_hbm,
                   posb, wvb, rowb, sem, sem2):
    c = lax.axis_index("c")
    s = lax.axis_index("s")
    w = s * NC + c
    pltpu.sync_copy(pos_hbm.at[w], posb)
    pltpu.sync_copy(w_hbm.at[w], wvb)
    xrow = (w & (NS - 1)) * _PPW
    pltpu.sync_copy(x_hbm.at[pl.ds(xrow, _PPW)], rowb)
    cp1 = pltpu.async_copy(rowb, xs_hbm.at[posb], sem)
    cp2 = pltpu.async_copy(wvb, wso_hbm.at[posb], sem2)
    cp1.wait()
    cp2.wait()


@functools.lru_cache(maxsize=None)
def _dispatch_kernel():
    return functools.partial(
        pl.kernel,
        out_type=(
            jax.ShapeDtypeStruct((P, D), jnp.float32),
            jax.ShapeDtypeStruct((P,), jnp.float32),
        ),
        mesh=plsc.VectorSubcoreMesh(core_axis_name="c", subcore_axis_name="s"),
        scratch_types=(
            pltpu.VMEM((_PPW,), jnp.int32),
            pltpu.VMEM((_PPW,), jnp.float32),
            pltpu.VMEM((_PPW, D), jnp.float32),
            pltpu.SemaphoreType.DMA,
            pltpu.SemaphoreType.DMA,
        ),
    )(_dispatch_body)


_HD = D // 2
_HF = F // 2


def _ffn_body(sp_ref, xs_ref, w1a_ref, w1b_ref, b1_ref, w2a_ref, w2b_ref,
              b2_ref, ws_ref, out_ref):
    i = pl.program_id(0)
    na = sp_ref[NT]

    @pl.when(i < na)
    def _():
        xb = xs_ref[...].astype(jnp.bfloat16)
        h = jnp.dot(xb[:, :_HD], w1a_ref[0].astype(jnp.bfloat16),
                    preferred_element_type=jnp.float32)
        h = h + jnp.dot(xb[:, _HD:], w1b_ref[0].astype(jnp.bfloat16),
                        preferred_element_type=jnp.float32)
        h = jnp.maximum(h + b1_ref[0], 0.0).astype(jnp.bfloat16)
        y = jnp.dot(h[:, :_HF], w2a_ref[0].astype(jnp.bfloat16),
                    preferred_element_type=jnp.float32)
        y = y + jnp.dot(h[:, _HF:], w2b_ref[0].astype(jnp.bfloat16),
                        preferred_element_type=jnp.float32)
        y = y + b2_ref[0]
        out_ref[...] = y * ws_ref[0]


def _ffn(sp, xs, W1, b1, W2, b2, wsr, interpret=False):
    grid_spec = pltpu.PrefetchScalarGridSpec(
        num_scalar_prefetch=1,
        grid=(NT,),
        in_specs=[
            pl.BlockSpec((BM, D), lambda i, sp: (i, 0)),
            pl.BlockSpec((1, _HD, F), lambda i, sp: (sp[i], 0, 0)),
            pl.BlockSpec((1, _HD, F), lambda i, sp: (sp[i], 1, 0)),
            pl.BlockSpec((1, 1, F), lambda i, sp: (sp[i], 0, 0)),
            pl.BlockSpec((1, _HF, D), lambda i, sp: (sp[i], 0, 0)),
            pl.BlockSpec((1, _HF, D), lambda i, sp: (sp[i], 1, 0)),
            pl.BlockSpec((1, 1, D), lambda i, sp: (sp[i], 0, 0)),
            pl.BlockSpec((1, BM, 1), lambda i, sp: (i, 0, 0)),
        ],
        out_specs=pl.BlockSpec((BM, D), lambda i, sp: (i, 0)),
    )
    return pl.pallas_call(
        _ffn_body,
        grid_spec=grid_spec,
        out_shape=jax.ShapeDtypeStruct((P, D), jnp.float32),
        interpret=interpret,
    )(sp, xs, W1, W1, b1.reshape(E, 1, F), W2, W2,
      b2.reshape(E, 1, D), wsr)


_TPW = T // NW


def _combine_body(ys_hbm, p0_hbm, p1_hbm, out_hbm, i0b, i1b, b0, b1, sem):
    c = lax.axis_index("c")
    s = lax.axis_index("s")
    w = s * NC + c
    pltpu.sync_copy(p0_hbm.at[w], i0b)
    pltpu.async_copy(ys_hbm.at[i0b], b0, sem).wait()
    pltpu.sync_copy(p1_hbm.at[w], i1b)
    pltpu.async_copy(ys_hbm.at[i1b], b1, sem).wait()

    def addrow(r, carry):
        for cc in range(D // 16):
            b0[r, pl.ds(cc * 16, 16)] = (
                b0[r, pl.ds(cc * 16, 16)] + b1[r, pl.ds(cc * 16, 16)])
        return carry

    lax.fori_loop(0, _TPW, addrow, 0)
    pltpu.sync_copy(b0, out_hbm.at[pl.ds(w * _TPW, _TPW)])


@functools.lru_cache(maxsize=None)
def _combine_kernel():
    return functools.partial(
        pl.kernel,
        out_type=jax.ShapeDtypeStruct((T, D), jnp.float32),
        mesh=plsc.VectorSubcoreMesh(core_axis_name="c", subcore_axis_name="s"),
        scratch_types=(
            pltpu.VMEM((_TPW,), jnp.int32),
            pltpu.VMEM((_TPW,), jnp.int32),
            pltpu.VMEM((_TPW, D), jnp.float32),
            pltpu.VMEM((_TPW, D), jnp.float32),
            pltpu.SemaphoreType.DMA,
        ),
    )(_combine_body)


def kernel(x, Wg, W1, b1, W2, b2):
    wg_pad = jnp.pad(Wg, ((0, 0), (0, 128 - E)))
    pos, wts, meta = _router(x, wg_pad)
    pos_flat = jnp.concatenate([pos[:, 0], pos[:, 1]]).reshape(K * T // 128, 128)
    w_flat = jnp.concatenate([wts[:, 0], wts[:, 1]]).reshape(K * T // 128, 128)
    sp = meta[:NT + 1, 0]

    xs, wso = _dispatch_kernel()(pos_flat, w_flat, x)
    ys = _ffn(sp, xs, W1, b1, W2, b2, wso.reshape(NT, BM, 1))
    out = _combine_kernel()(
        ys, pos[:, 0].reshape(NW, _TPW), pos[:, 1].reshape(NW, _TPW))
    return out


# speedup vs baseline: 1.1522x; 1.0195x over previous
import functools

import jax
import jax.numpy as jnp
from jax import lax
from jax.experimental import pallas as pl
from jax.experimental.pallas import tpu as pltpu
from jax.experimental.pallas import tpu_sc as plsc

E = 8
K = 2
D = 768
F = 3072
T = 2048
BM = 256
NT = 24
P = NT * BM
NC = 2
NS = 16
NW = NC * NS

_NEG = -1e30


def _router_body(x_ref, wg_ref, pos0_ref, pos1_ref, w0_ref, w1_ref, meta_ref,
                 oh_ref, cum_ref):
    xl = x_ref[...]
    wg = wg_ref[...]
    logits = jnp.dot(xl, wg, preferred_element_type=jnp.float32)
    col = lax.broadcasted_iota(jnp.int32, (T, 128), 1)
    lg = jnp.where(col < E, logits, _NEG)
    m1 = jnp.max(lg, axis=1, keepdims=True)
    a1 = jnp.min(jnp.where(lg == m1, col, 128), axis=1, keepdims=True)
    lg2 = jnp.where(col == a1, _NEG, lg)
    m2 = jnp.max(lg2, axis=1, keepdims=True)
    a2 = jnp.min(jnp.where(lg2 == m2, col, 128), axis=1, keepdims=True)
    w1v = 1.0 / (1.0 + jnp.exp(m2 - m1))
    w2v = 1.0 - w1v

    oh_ref[...] = ((col == a1) | (col == a2)).astype(jnp.float32)
    ri = lax.broadcasted_iota(jnp.int32, (128, 128), 0)
    ci = lax.broadcasted_iota(jnp.int32, (128, 128), 1)
    lower = (ri > ci).astype(jnp.float32)
    carry = jnp.zeros((1, 128), jnp.float32)
    for c in range(T // 128):
        chunk = oh_ref[pl.ds(c * 128, 128), :]
        cum_ref[pl.ds(c * 128, 128), :] = (
            jnp.dot(lower, chunk, preferred_element_type=jnp.float32) + carry)
        carry = carry + jnp.sum(chunk, axis=0, keepdims=True)

    cnt = carry.astype(jnp.int32)
    q = (cnt + (BM - 1)) // BM
    upper = (ri < ci).astype(jnp.float32)
    offq = jnp.dot(q.astype(jnp.float32), upper,
                   preferred_element_type=jnp.float32)
    off = offq * float(BM)
    na = jnp.sum(q)

    cum = cum_ref[...]
    offb = jnp.broadcast_to(off, (T, 128))
    p1 = jnp.sum(jnp.where(col == a1, cum + offb, 0.0), axis=1, keepdims=True)
    p2 = jnp.sum(jnp.where(col == a2, cum + offb, 0.0), axis=1, keepdims=True)

    pos0_ref[...] = p1.astype(jnp.int32)
    pos1_ref[...] = p2.astype(jnp.int32)
    w0_ref[...] = w1v
    w1_ref[...] = w2v

    r32 = lax.broadcasted_iota(jnp.int32, (32, 128), 0)
    c32 = lax.broadcasted_iota(jnp.int32, (32, 128), 1)
    starts = (r32 * BM).astype(jnp.float32)
    off32 = jnp.broadcast_to(off, (32, 128))
    ind = ((starts >= off32) & (c32 >= 1) & (c32 < E)).astype(jnp.int32)
    et = jnp.sum(ind, axis=1, keepdims=True)
    meta_ref[...] = jnp.where(r32 == NT, na, jnp.broadcast_to(et, (32, 128)))


def _router(x, wg_pad, interpret=False):
    return pl.pallas_call(
        _router_body,
        out_shape=(
            jax.ShapeDtypeStruct((T, 1), jnp.int32),
            jax.ShapeDtypeStruct((T, 1), jnp.int32),
            jax.ShapeDtypeStruct((T, 1), jnp.float32),
            jax.ShapeDtypeStruct((T, 1), jnp.float32),
            jax.ShapeDtypeStruct((32, 128), jnp.int32),
        ),
        scratch_shapes=[
            pltpu.VMEM((T, 128), jnp.float32),
            pltpu.VMEM((T, 128), jnp.float32),
        ],
        interpret=interpret,
    )(x, wg_pad)


_PPW = (K * T) // NW


_HCH = _PPW // 2


def _dispatch_body(pos_hbm, w_hbm, x_hbm, xs_hbm, wso_hbm,
                   posb, wvb, rb0, rb1, si, sw, sx0, sx1, ss0, ss1):
    c = lax.axis_index("c")
    s = lax.axis_index("s")
    w = s * NC + c
    cpp = pltpu.async_copy(pos_hbm.at[w], posb, si)
    cpw = pltpu.async_copy(w_hbm.at[w], wvb, sw)
    xrow = (w & (NS - 1)) * _PPW
    cx0 = pltpu.async_copy(x_hbm.at[pl.ds(xrow, _HCH)], rb0, sx0)
    cx1 = pltpu.async_copy(x_hbm.at[pl.ds(xrow + _HCH, _HCH)], rb1, sx1)
    cpp.wait()
    cpw.wait()
    cx0.wait()
    s0 = pltpu.async_copy(rb0, xs_hbm.at[posb.at[0]], ss0)
    w0 = pltpu.async_copy(wvb.at[0], wso_hbm.at[posb.at[0]], sw)
    cx1.wait()
    s1 = pltpu.async_copy(rb1, xs_hbm.at[posb.at[1]], ss1)
    w1 = pltpu.async_copy(wvb.at[1], wso_hbm.at[posb.at[1]], sw)
    s0.wait()
    w0.wait()
    s1.wait()
    w1.wait()


@functools.lru_cache(maxsize=None)
def _dispatch_kernel():
    return functools.partial(
        pl.kernel,
        out_type=(
            jax.ShapeDtypeStruct((P, D), jnp.float32),
            jax.ShapeDtypeStruct((P,), jnp.float32),
        ),
        mesh=plsc.VectorSubcoreMesh(core_axis_name="c", subcore_axis_name="s"),
        scratch_types=(
            pltpu.VMEM((2, _HCH), jnp.int32),
            pltpu.VMEM((2, _HCH), jnp.float32),
            pltpu.VMEM((_HCH, D), jnp.float32),
            pltpu.VMEM((_HCH, D), jnp.float32),
            pltpu.SemaphoreType.DMA,
            pltpu.SemaphoreType.DMA,
            pltpu.SemaphoreType.DMA,
            pltpu.SemaphoreType.DMA,
            pltpu.SemaphoreType.DMA,
            pltpu.SemaphoreType.DMA,
        ),
    )(_dispatch_body)


_HD = D // 2
_HF = F // 2


def _ffn_body(sp_ref, xs_ref, w1a_ref, w1b_ref, b1_ref, w2a_ref, w2b_ref,
              b2_ref, ws_ref, out_ref):
    i = pl.program_id(0)
    na = sp_ref[NT]

    @pl.when(i < na)
    def _():
        xb = xs_ref[...].astype(jnp.bfloat16)
        h = jnp.dot(xb[:, :_HD], w1a_ref[0].astype(jnp.bfloat16),
                    preferred_element_type=jnp.float32)
        h = h + jnp.dot(xb[:, _HD:], w1b_ref[0].astype(jnp.bfloat16),
                        preferred_element_type=jnp.float32)
        h = jnp.maximum(h + b1_ref[0], 0.0).astype(jnp.bfloat16)
        y = jnp.dot(h[:, :_HF], w2a_ref[0].astype(jnp.bfloat16),
                    preferred_element_type=jnp.float32)
        y = y + jnp.dot(h[:, _HF:], w2b_ref[0].astype(jnp.bfloat16),
                        preferred_element_type=jnp.float32)
        y = y + b2_ref[0]
        out_ref[...] = y * ws_ref[0]


def _ffn(sp, xs, W1, b1, W2, b2, wsr, interpret=False):
    grid_spec = pltpu.PrefetchScalarGridSpec(
        num_scalar_prefetch=1,
        grid=(NT,),
        in_specs=[
            pl.BlockSpec((BM, D), lambda i, sp: (i, 0)),
            pl.BlockSpec((1, _HD, F), lambda i, sp: (sp[i], 0, 0)),
            pl.BlockSpec((1, _HD, F), lambda i, sp: (sp[i], 1, 0)),
            pl.BlockSpec((1, 1, F), lambda i, sp: (sp[i], 0, 0)),
            pl.BlockSpec((1, _HF, D), lambda i, sp: (sp[i], 0, 0)),
            pl.BlockSpec((1, _HF, D), lambda i, sp: (sp[i], 1, 0)),
            pl.BlockSpec((1, 1, D), lambda i, sp: (sp[i], 0, 0)),
            pl.BlockSpec((1, BM, 1), lambda i, sp: (i, 0, 0)),
        ],
        out_specs=pl.BlockSpec((BM, D), lambda i, sp: (i, 0)),
    )
    return pl.pallas_call(
        _ffn_body,
        grid_spec=grid_spec,
        out_shape=jax.ShapeDtypeStruct((P, D), jnp.float32),
        interpret=interpret,
    )(sp, xs, W1, W1, b1.reshape(E, 1, F), W2, W2,
      b2.reshape(E, 1, D), wsr)


_TPW = T // NW


def _combine_body(ys_hbm, p0_hbm, p1_hbm, out_hbm, i0b, i1b, b0, b1, sem):
    c = lax.axis_index("c")
    s = lax.axis_index("s")
    w = s * NC + c
    pltpu.sync_copy(p0_hbm.at[w], i0b)
    pltpu.async_copy(ys_hbm.at[i0b], b0, sem).wait()
    pltpu.sync_copy(p1_hbm.at[w], i1b)
    pltpu.async_copy(ys_hbm.at[i1b], b1, sem).wait()

    def addrow(r, carry):
        for cc in range(D // 16):
            b0[r, pl.ds(cc * 16, 16)] = (
                b0[r, pl.ds(cc * 16, 16)] + b1[r, pl.ds(cc * 16, 16)])
        return carry

    lax.fori_loop(0, _TPW, addrow, 0)
    pltpu.sync_copy(b0, out_hbm.at[pl.ds(w * _TPW, _TPW)])


@functools.lru_cache(maxsize=None)
def _combine_kernel():
    return functools.partial(
        pl.kernel,
        out_type=jax.ShapeDtypeStruct((T, D), jnp.float32),
        mesh=plsc.VectorSubcoreMesh(core_axis_name="c", subcore_axis_name="s"),
        scratch_types=(
            pltpu.VMEM((_TPW,), jnp.int32),
            pltpu.VMEM((_TPW,), jnp.int32),
            pltpu.VMEM((_TPW, D), jnp.float32),
            pltpu.VMEM((_TPW, D), jnp.float32),
            pltpu.SemaphoreType.DMA,
        ),
    )(_combine_body)


def kernel(x, Wg, W1, b1, W2, b2):
    wg_pad = jnp.pad(Wg, ((0, 0), (0, 128 - E)))
    pos0, pos1, w0, w1, meta = _router(x, wg_pad)
    pos_in = jnp.concatenate([pos0.reshape(NW // 2, 2, _HCH),
                              pos1.reshape(NW // 2, 2, _HCH)], axis=0)
    w_in = jnp.concatenate([w0.reshape(NW // 2, 2, _HCH),
                            w1.reshape(NW // 2, 2, _HCH)], axis=0)
    sp = meta[:NT + 1, 0]

    xs, wso = _dispatch_kernel()(pos_in, w_in, x)
    ys = _ffn(sp, xs, W1, b1, W2, b2, wso.reshape(NT, BM, 1))
    out = _combine_kernel()(
        ys, pos0.reshape(NW, _TPW), pos1.reshape(NW, _TPW))
    return out
